# baseline (speedup 1.0000x reference)
import functools

import jax
import jax.numpy as jnp
from jax import lax
from jax.experimental import pallas as pl
from jax.experimental.pallas import tpu as pltpu
from jax.experimental.pallas import tpu_sc as plsc

F32 = jnp.float32
D = 48
B = 64
NP = 5
FUT = 40
BT = B * NP
MEM = 50000
MBLK = 2000
NBLK = MEM // MBLK
HW = 90
PIX = HW * HW

SC_NC, SC_NS = 2, 16
NW = SC_NC * SC_NS


def _sigmoid(x):
    return jax.nn.sigmoid(x)


def _sc_gather(table, idx1d, total, chunk, dfeat):
    per_w = total // NW
    nchunk = per_w // chunk
    mesh = plsc.VectorSubcoreMesh(core_axis_name="c", subcore_axis_name="s")

    @functools.partial(
        pl.kernel,
        mesh=mesh,
        out_type=jax.ShapeDtypeStruct((total, dfeat), F32),
        scratch_types=[
            pltpu.VMEM((per_w,), jnp.int32),
            pltpu.VMEM((per_w, dfeat), F32),
            pltpu.SemaphoreType.DMA,
        ],
    )
    def k(table_hbm, idx_hbm, out_hbm, idx_v, rows_v, sem):
        wid = lax.axis_index("s") * SC_NC + lax.axis_index("c")
        base = wid * per_w
        pltpu.sync_copy(idx_hbm.at[pl.ds(base, per_w)], idx_v)
        copies = []
        for j in range(nchunk):
            copies.append(
                pltpu.async_copy(
                    table_hbm.at[idx_v.at[pl.ds(j * chunk, chunk)]],
                    rows_v.at[pl.ds(j * chunk, chunk)], sem)
            )
        for c in copies:
            c.wait()
        pltpu.sync_copy(rows_v, out_hbm.at[pl.ds(base, per_w)])

    return k(table, idx1d)


def _enc_kernel(pat_ref, w6_ref, b16_ref,
                wir_ref, wiz_ref, win_ref, whr_ref, whz_ref, whn_ref,
                bir_ref, biz_ref, bin_ref, bhr_ref, bhz_ref, bhn_ref,
                h_out, q_out, story_ref):
    pat = pat_ref[...].reshape(20 * B, 6)
    story = jax.nn.relu(
        jnp.dot(pat, w6_ref[...], preferred_element_type=F32) + b16_ref[...]
    )
    story_ref[...] = story.reshape(20, B, 16)

    def step(t, h):
        x = story_ref[t]
        gr = jnp.dot(x, wir_ref[...], preferred_element_type=F32) + bir_ref[...] \
            + jnp.dot(h, whr_ref[...], preferred_element_type=F32) + bhr_ref[...]
        gz = jnp.dot(x, wiz_ref[...], preferred_element_type=F32) + biz_ref[...] \
            + jnp.dot(h, whz_ref[...], preferred_element_type=F32) + bhz_ref[...]
        r = _sigmoid(gr)
        z = _sigmoid(gz)
        hn_pre = jnp.dot(h, whn_ref[...], preferred_element_type=F32) + bhn_ref[...]
        n = jnp.tanh(jnp.dot(x, win_ref[...], preferred_element_type=F32)
                     + bin_ref[...] + r * hn_pre)
        return (1.0 - z) * n + z * h

    h = lax.fori_loop(0, 20, step, jnp.zeros((B, D), F32))
    h_out[...] = h
    nrm = jnp.sqrt(jnp.sum(h * h, axis=1, keepdims=True))
    q_out[...] = h / jnp.maximum(nrm, 1e-12)


def _scene_kernel(ph_ref, w1_ref, b1_ref, w2_ref, b2_ref, out_ref, s1p_ref):
    pieces = []
    for dy in range(5):
        ay, by = dy // 2, dy % 2
        for dx in range(5):
            ax, bx = dx // 2, dx % 2
            sl = ph_ref[0, by, bx, ay:ay + HW, ax:ax + HW, :]
            pieces.append(sl.reshape(PIX, 4))
    patches = jnp.concatenate(pieces, axis=1)
    s1 = jax.nn.relu(
        jnp.dot(patches, w1_ref[...], preferred_element_type=F32) + b1_ref[...]
    )
    s1p_ref[...] = jnp.zeros((94, 94, 16), F32)
    s1p_ref[2:92, 2:92, :] = s1.reshape(HW, HW, 16)
    acc = jnp.zeros((PIX, 32), F32) + b2_ref[...]
    for dy in range(5):
        row_pieces = []
        for dx in range(5):
            row_pieces.append(s1p_ref[dy:dy + HW, dx:dx + HW, :].reshape(PIX, 16))
        patch2 = jnp.concatenate(row_pieces, axis=1)
        acc = acc + jnp.dot(patch2, w2_ref[dy], preferred_element_type=F32)
    out_ref[0] = jnp.concatenate(
        [jax.nn.relu(acc), jnp.zeros((PIX, 96), F32)], axis=1)


def _sim_kernel(q_ref, mem_ref, vals_ref, idx_ref):
    m = mem_ref[...]
    ss = jnp.sum(m * m, axis=1, keepdims=True)
    mn = m / jnp.maximum(jnp.sqrt(ss), 1e-12)
    s = lax.dot_general(q_ref[...], mn, (((1,), (1,)), ((), ())),
                        preferred_element_type=F32)
    iota = lax.broadcasted_iota(jnp.int32, (B, MBLK), 1)
    base = pl.program_id(0) * MBLK
    vs, ids = [], []
    for _ in range(NP):
        mx = jnp.max(s, axis=1, keepdims=True)
        is_m = s == mx
        am = jnp.min(jnp.where(is_m, iota, jnp.int32(1 << 30)), axis=1,
                     keepdims=True)
        vs.append(mx)
        ids.append(am + base)
        s = jnp.where(iota == am, -jnp.inf, s)
    vals_ref[0] = jnp.concatenate(vs, axis=1)
    idx_ref[0] = jnp.concatenate(ids, axis=1)


def _merge_kernel(cv_ref, ci_ref, idx_ref):
    s = cv_ref[...]
    ci = ci_ref[...]
    ncand = NBLK * NP
    iota = lax.broadcasted_iota(jnp.int32, (B, ncand), 1)
    ids = []
    for _ in range(NP):
        mx = jnp.max(s, axis=1, keepdims=True)
        is_m = s == mx
        p = jnp.min(jnp.where(is_m, iota, jnp.int32(1 << 30)), axis=1,
                    keepdims=True)
        hit = iota == p
        ids.append(jnp.sum(jnp.where(hit, ci, 0), axis=1, keepdims=True))
        s = jnp.where(hit, -jnp.inf, s)
    idx_ref[...] = jnp.concatenate(ids, axis=1)


def _pred_to_idx(p):
    ix = (p[:, :, 0:1] + 89.0) * 0.5
    iy = (p[:, :, 1:2] + 89.0) * 0.5
    ixn = jnp.round(ix).astype(jnp.int32)
    iyn = jnp.round(iy).astype(jnp.int32)
    valid = ((ixn >= 0) & (ixn < HW) & (iyn >= 0) & (iyn < HW))
    ixc = jnp.clip(ixn, 0, HW - 1)
    iyc = jnp.clip(iyn, 0, HW - 1)
    boff = (lax.broadcasted_iota(jnp.int32, (FUT, BT, 1), 1) // NP) * PIX
    flat = boff + iyc * HW + ixc
    return flat, valid.astype(F32)


def _dec_kernel(inp_ref, c0_ref,
                wir_ref, wiz_ref, win_ref, whr_ref, whz_ref, whn_ref,
                bir_ref, biz_ref, bin_ref, bhr_ref, bhz_ref, bhn_ref,
                wfc_ref, bfc_ref,
                pred_ref, idx_ref, val_ref):
    inp = inp_ref[...]
    gi_r0 = jnp.dot(inp, wir_ref[...], preferred_element_type=F32) + bir_ref[...]
    gi_z0 = jnp.dot(inp, wiz_ref[...], preferred_element_type=F32) + biz_ref[...]
    gi_n0 = jnp.dot(inp, win_ref[...], preferred_element_type=F32) + bin_ref[...]

    def cell(h, gi_r, gi_z, gi_n):
        gr = gi_r + jnp.dot(h, whr_ref[...], preferred_element_type=F32) + bhr_ref[...]
        gz = gi_z + jnp.dot(h, whz_ref[...], preferred_element_type=F32) + bhz_ref[...]
        r = _sigmoid(gr)
        z = _sigmoid(gz)
        hn_pre = jnp.dot(h, whn_ref[...], preferred_element_type=F32) + bhn_ref[...]
        n = jnp.tanh(gi_n + r * hn_pre)
        return (1.0 - z) * n + z * h

    h0 = jnp.zeros((BT, 2 * D), F32)
    h = cell(h0, gi_r0, gi_z0, gi_n0)
    c = c0_ref[...] + jnp.dot(h, wfc_ref[...], preferred_element_type=F32) + bfc_ref[...]
    pred_ref[0] = c

    def step(t, carry):
        h, c = carry
        h = cell(h, bir_ref[...], biz_ref[...], bin_ref[...])
        c = c + jnp.dot(h, wfc_ref[...], preferred_element_type=F32) + bfc_ref[...]
        pred_ref[t] = c
        return (h, c)

    lax.fori_loop(1, FUT, step, (h, c))
    p = pred_ref[...]
    flat, valid = _pred_to_idx(p)
    idx_ref[...] = flat
    val_ref[...] = valid


def _refine_kernel(pred_in_ref, rows_ref, valm_ref, h0_ref,
                   wir_ref, wiz_ref, win_ref, whr_ref, whz_ref, whn_ref,
                   bir_ref, biz_ref, bin_ref, bhr_ref, bhz_ref, bhn_ref,
                   wr3_ref, br3_ref,
                   pred_ref, idx_ref, val_ref):
    def step(t, h):
        x = rows_ref[t] * valm_ref[t]
        gr = jnp.dot(x, wir_ref[...], preferred_element_type=F32) + bir_ref[...] \
            + jnp.dot(h, whr_ref[...], preferred_element_type=F32) + bhr_ref[...]
        gz = jnp.dot(x, wiz_ref[...], preferred_element_type=F32) + biz_ref[...] \
            + jnp.dot(h, whz_ref[...], preferred_element_type=F32) + bhz_ref[...]
        r = _sigmoid(gr)
        z = _sigmoid(gz)
        hn_pre = jnp.dot(h, whn_ref[...], preferred_element_type=F32) + bhn_ref[...]
        n = jnp.tanh(jnp.dot(x, win_ref[...], preferred_element_type=F32)
                     + bin_ref[...] + r * hn_pre)
        return (1.0 - z) * n + z * h

    h = lax.fori_loop(0, FUT, step, h0_ref[...])

    def wstep(t, _):
        r_t = jnp.dot(h, wr3_ref[t], preferred_element_type=F32) + br3_ref[t]
        pred_ref[t] = pred_in_ref[t] + r_t
        return 0

    lax.fori_loop(0, FUT, wstep, 0)
    p = pred_ref[...]
    flat, valid = _pred_to_idx(p)
    idx_ref[...] = flat
    val_ref[...] = valid


def _split3(w, n):
    return w[0:n].T, w[n:2 * n].T, w[2 * n:3 * n].T


def _b3(b, n):
    return b[0:n].reshape(1, n), b[n:2 * n].reshape(1, n), b[2 * n:3 * n].reshape(1, n)


def kernel(past, scene, memory_past, memory_fut, params):
    p = params

    w6 = jnp.transpose(p['W_conv_past'], (2, 1, 0)).reshape(6, 16)
    b16 = p['b_conv_past'].reshape(1, 16)
    e_wir, e_wiz, e_win = _split3(p['W_ih_enc'], D)
    e_whr, e_whz, e_whn = _split3(p['W_hh_enc'], D)
    e_bir, e_biz, e_bin = _b3(p['b_ih_enc'], D)
    e_bhr, e_bhz, e_bhn = _b3(p['b_hh_enc'], D)
    d_wir, d_wiz, d_win = _split3(p['W_ih_dec'], 2 * D)
    d_whr, d_whz, d_whn = _split3(p['W_hh_dec'], 2 * D)
    d_bir, d_biz, d_bin = _b3(p['b_ih_dec'], 2 * D)
    d_bhr, d_bhz, d_bhn = _b3(p['b_hh_dec'], 2 * D)
    s_wir, s_wiz, s_win = _split3(p['W_ih_scn'], D)
    s_whr, s_whz, s_whn = _split3(p['W_hh_scn'], D)
    s_bir, s_biz, s_bin = _b3(p['b_ih_scn'], D)
    s_bhr, s_bhz, s_bhn = _b3(p['b_hh_scn'], D)
    wfc = p['W_fc_out'].T
    bfc = p['b_fc_out'].reshape(1, 2)
    w1 = jnp.transpose(p['W_cs1'], (2, 3, 1, 0)).reshape(100, 16)
    b1 = p['b_cs1'].reshape(1, 16)
    w2 = jnp.transpose(p['W_cs2'], (2, 3, 1, 0)).reshape(5, 80, 32)
    b2 = p['b_cs2'].reshape(1, 32)
    wr3 = jnp.transpose(p['W_refine'].reshape(FUT, 2, D), (0, 2, 1))
    br3 = p['b_refine'].reshape(FUT, 1, 2)

    xt = jnp.transpose(past, (1, 0, 2))
    xp = jnp.concatenate([jnp.zeros((1, B, 2), F32), xt, jnp.zeros((1, B, 2), F32)], 0)
    pat = jnp.concatenate([xp[0:20], xp[1:21], xp[2:22]], axis=2)

    state_past, q_n = pl.pallas_call(
        _enc_kernel,
        out_shape=[jax.ShapeDtypeStruct((B, D), F32),
                   jax.ShapeDtypeStruct((B, D), F32)],
        scratch_shapes=[pltpu.VMEM((20, B, 16), F32)],
    )(pat, w6, b16, e_wir, e_wiz, e_win, e_whr, e_whz, e_whn,
      e_bir, e_biz, e_bin, e_bhr, e_bhz, e_bhn)

    sp = jnp.pad(scene, ((0, 0), (2, 2), (2, 2), (0, 0)))
    phases = jnp.transpose(sp.reshape(B, 92, 2, 92, 2, 4), (0, 2, 4, 1, 3, 5))
    s2 = pl.pallas_call(
        _scene_kernel,
        grid=(B,),
        in_specs=[
            pl.BlockSpec((1, 2, 2, 92, 92, 4), lambda b: (b, 0, 0, 0, 0, 0)),
            pl.BlockSpec((100, 16), lambda b: (0, 0)),
            pl.BlockSpec((1, 16), lambda b: (0, 0)),
            pl.BlockSpec((5, 80, 32), lambda b: (0, 0, 0)),
            pl.BlockSpec((1, 32), lambda b: (0, 0)),
        ],
        out_specs=pl.BlockSpec((1, PIX, 128), lambda b: (b, 0, 0)),
        out_shape=jax.ShapeDtypeStruct((B, PIX, 128), F32),
        scratch_shapes=[pltpu.VMEM((94, 94, 16), F32)],
    )(phases, w1, b1, w2, b2)
    table = s2.reshape(B * PIX, 128)

    bvals, bidx = pl.pallas_call(
        _sim_kernel,
        grid=(NBLK,),
        in_specs=[
            pl.BlockSpec((B, D), lambda i: (0, 0)),
            pl.BlockSpec((MBLK, D), lambda i: (i, 0)),
        ],
        out_specs=[
            pl.BlockSpec((1, B, NP), lambda i: (i, 0, 0)),
            pl.BlockSpec((1, B, NP), lambda i: (i, 0, 0)),
        ],
        out_shape=[jax.ShapeDtypeStruct((NBLK, B, NP), F32),
                   jax.ShapeDtypeStruct((NBLK, B, NP), jnp.int32)],
    )(q_n, memory_past)
    cv = jnp.transpose(bvals, (1, 0, 2)).reshape(B, NBLK * NP)
    ci = jnp.transpose(bidx, (1, 0, 2)).reshape(B, NBLK * NP)
    topk = pl.pallas_call(
        _merge_kernel,
        out_shape=jax.ShapeDtypeStruct((B, NP), jnp.int32),
    )(cv, ci)

    fidx = jnp.concatenate([topk.reshape(BT), jnp.zeros((512 - BT,), jnp.int32)])
    fut_pad = jnp.pad(memory_fut, ((0, 0), (0, 128 - D)))
    fut_rows = _sc_gather(fut_pad, fidx, 512, 16, 128)
    info_future = fut_rows.reshape(512, 128)[:BT, :D]

    state_rep = jnp.repeat(state_past, NP, axis=0)
    inp0 = jnp.concatenate([state_rep, info_future], axis=1)
    c0 = jnp.repeat(past[:, -1, :], NP, axis=0)
    pred, idxf, valf = pl.pallas_call(
        _dec_kernel,
        out_shape=[jax.ShapeDtypeStruct((FUT, BT, 2), F32),
                   jax.ShapeDtypeStruct((FUT, BT, 1), jnp.int32),
                   jax.ShapeDtypeStruct((FUT, BT, 1), F32)],
    )(inp0, c0, d_wir, d_wiz, d_win, d_whr, d_whz, d_whn,
      d_bir, d_biz, d_bin, d_bhr, d_bhz, d_bhn, wfc, bfc)

    for _ in range(4):
        rows = _sc_gather(table, idxf.reshape(FUT * BT), FUT * BT, 80, 128)
        rows = rows.reshape(FUT, BT, 128)[:, :, :32]
        pred, idxf, valf = pl.pallas_call(
            _refine_kernel,
            out_shape=[jax.ShapeDtypeStruct((FUT, BT, 2), F32),
                       jax.ShapeDtypeStruct((FUT, BT, 1), jnp.int32),
                       jax.ShapeDtypeStruct((FUT, BT, 1), F32)],
        )(pred, rows, valf, state_rep,
          s_wir, s_wiz, s_win, s_whr, s_whz, s_whn,
          s_bir, s_biz, s_bin, s_bhr, s_bhz, s_bhn, wr3, br3)

    return jnp.transpose(pred, (1, 0, 2)).reshape(B, NP, FUT, 2)


# speedup vs baseline: 1.0265x; 1.0265x over previous
import functools

import jax
import jax.numpy as jnp
from jax import lax
from jax.experimental import pallas as pl
from jax.experimental.pallas import tpu as pltpu
from jax.experimental.pallas import tpu_sc as plsc

F32 = jnp.float32
D = 48
B = 64
NP = 5
FUT = 40
BT = B * NP
MEM = 50000
MBLK = 2000
NBLK = MEM // MBLK
HW = 90
PIX = HW * HW
PIXP = PIX + 4

SC_NC, SC_NS = 2, 16
NW = SC_NC * SC_NS


def _sigmoid(x):
    return jax.nn.sigmoid(x)


def _sc_gather(table, idx1d, total, chunk, dfeat):
    per_w = total // NW
    nchunk = per_w // chunk
    mesh = plsc.VectorSubcoreMesh(core_axis_name="c", subcore_axis_name="s")

    @functools.partial(
        pl.kernel,
        mesh=mesh,
        out_type=jax.ShapeDtypeStruct((total, dfeat), F32),
        scratch_types=[
            pltpu.VMEM((per_w,), jnp.int32),
            pltpu.VMEM((per_w, dfeat), F32),
            pltpu.SemaphoreType.DMA,
        ],
    )
    def k(table_hbm, idx_hbm, out_hbm, idx_v, rows_v, sem):
        wid = lax.axis_index("s") * SC_NC + lax.axis_index("c")
        base = wid * per_w
        pltpu.sync_copy(idx_hbm.at[pl.ds(base, per_w)], idx_v)
        copies = []
        for j in range(nchunk):
            copies.append(
                pltpu.async_copy(
                    table_hbm.at[idx_v.at[pl.ds(j * chunk, chunk)]],
                    rows_v.at[pl.ds(j * chunk, chunk)], sem)
            )
        for c in copies:
            c.wait()
        pltpu.sync_copy(rows_v, out_hbm.at[pl.ds(base, per_w)])

    return k(table, idx1d)


def _enc_kernel(pat_ref, w6_ref, b16_ref,
                wir_ref, wiz_ref, win_ref, whr_ref, whz_ref, whn_ref,
                bir_ref, biz_ref, bin_ref, bhr_ref, bhz_ref, bhn_ref,
                h_out, q_out, story_ref):
    pat = pat_ref[...].reshape(20 * B, 6)
    story = jax.nn.relu(
        jnp.dot(pat, w6_ref[...], preferred_element_type=F32) + b16_ref[...]
    )
    story_ref[...] = story.reshape(20, B, 16)

    def step(t, h):
        x = story_ref[t]
        gr = jnp.dot(x, wir_ref[...], preferred_element_type=F32) + bir_ref[...] \
            + jnp.dot(h, whr_ref[...], preferred_element_type=F32) + bhr_ref[...]
        gz = jnp.dot(x, wiz_ref[...], preferred_element_type=F32) + biz_ref[...] \
            + jnp.dot(h, whz_ref[...], preferred_element_type=F32) + bhz_ref[...]
        r = _sigmoid(gr)
        z = _sigmoid(gz)
        hn_pre = jnp.dot(h, whn_ref[...], preferred_element_type=F32) + bhn_ref[...]
        n = jnp.tanh(jnp.dot(x, win_ref[...], preferred_element_type=F32)
                     + bin_ref[...] + r * hn_pre)
        return (1.0 - z) * n + z * h

    h = lax.fori_loop(0, 20, step, jnp.zeros((B, D), F32))
    h_out[...] = h
    nrm = jnp.sqrt(jnp.sum(h * h, axis=1, keepdims=True))
    q_out[...] = h / jnp.maximum(nrm, 1e-12)


def _scene_kernel(ph_ref, w1_ref, b1_ref, w2_ref, b2_ref, out_ref, s1p_ref):
    pieces = []
    for dy in range(5):
        ay, by = dy // 2, dy % 2
        for dx in range(5):
            ax, bx = dx // 2, dx % 2
            sl = ph_ref[0, by, bx, ay:ay + HW, ax:ax + HW, :]
            pieces.append(sl.reshape(PIX, 4))
    patches = jnp.concatenate(pieces, axis=1)
    s1 = jax.nn.relu(
        jnp.dot(patches, w1_ref[...], preferred_element_type=F32) + b1_ref[...]
    )
    s1p_ref[...] = jnp.zeros((94, 94, 16), F32)
    s1p_ref[2:92, 2:92, :] = s1.reshape(HW, HW, 16)
    acc = jnp.zeros((PIX, 32), F32) + b2_ref[...]
    for dy in range(5):
        row_pieces = []
        for dx in range(5):
            row_pieces.append(s1p_ref[dy:dy + HW, dx:dx + HW, :].reshape(PIX, 16))
        patch2 = jnp.concatenate(row_pieces, axis=1)
        acc = acc + jnp.dot(patch2, w2_ref[dy], preferred_element_type=F32)
    out_ref[...] = jnp.concatenate(
        [jnp.concatenate([jax.nn.relu(acc), jnp.zeros((4, 32), F32)], axis=0),
         jnp.zeros((PIXP, 96), F32)], axis=1)


def _sim_kernel(q_ref, mem_ref, vals_ref, idx_ref):
    m = mem_ref[...]
    ss = jnp.sum(m * m, axis=1, keepdims=True)
    mn = m / jnp.maximum(jnp.sqrt(ss), 1e-12)
    s = lax.dot_general(q_ref[...], mn, (((1,), (1,)), ((), ())),
                        preferred_element_type=F32)
    iota = lax.broadcasted_iota(jnp.int32, (B, MBLK), 1)
    base = pl.program_id(0) * MBLK
    vs, ids = [], []
    for _ in range(NP):
        mx = jnp.max(s, axis=1, keepdims=True)
        is_m = s == mx
        am = jnp.min(jnp.where(is_m, iota, jnp.int32(1 << 30)), axis=1,
                     keepdims=True)
        vs.append(mx)
        ids.append(am + base)
        s = jnp.where(iota == am, -jnp.inf, s)
    vals_ref[0] = jnp.concatenate(vs, axis=1)
    idx_ref[0] = jnp.concatenate(ids, axis=1)


def _merge_kernel(cv_ref, ci_ref, idx_ref):
    s = cv_ref[...]
    ci = ci_ref[...]
    ncand = NBLK * NP
    iota = lax.broadcasted_iota(jnp.int32, (B, ncand), 1)
    ids = []
    for _ in range(NP):
        mx = jnp.max(s, axis=1, keepdims=True)
        is_m = s == mx
        p = jnp.min(jnp.where(is_m, iota, jnp.int32(1 << 30)), axis=1,
                    keepdims=True)
        hit = iota == p
        ids.append(jnp.sum(jnp.where(hit, ci, 0), axis=1, keepdims=True))
        s = jnp.where(hit, -jnp.inf, s)
    idx_ref[...] = jnp.concatenate(ids, axis=1)


def _pred_to_idx(p):
    ix = (p[:, :, 0:1] + 89.0) * 0.5
    iy = (p[:, :, 1:2] + 89.0) * 0.5
    ixn = jnp.round(ix).astype(jnp.int32)
    iyn = jnp.round(iy).astype(jnp.int32)
    valid = ((ixn >= 0) & (ixn < HW) & (iyn >= 0) & (iyn < HW))
    ixc = jnp.clip(ixn, 0, HW - 1)
    iyc = jnp.clip(iyn, 0, HW - 1)
    boff = (lax.broadcasted_iota(jnp.int32, (FUT, BT, 1), 1) // NP) * PIXP
    flat = boff + iyc * HW + ixc
    return flat, valid.astype(F32)


def _dec_kernel(inp_ref, c0_ref,
                wir_ref, wiz_ref, win_ref, whr_ref, whz_ref, whn_ref,
                bir_ref, biz_ref, bin_ref, bhr_ref, bhz_ref, bhn_ref,
                wfc_ref, bfc_ref,
                pred_ref, idx_ref, val_ref):
    inp = inp_ref[...]
    gi_r0 = jnp.dot(inp, wir_ref[...], preferred_element_type=F32) + bir_ref[...]
    gi_z0 = jnp.dot(inp, wiz_ref[...], preferred_element_type=F32) + biz_ref[...]
    gi_n0 = jnp.dot(inp, win_ref[...], preferred_element_type=F32) + bin_ref[...]

    def cell(h, gi_r, gi_z, gi_n):
        gr = gi_r + jnp.dot(h, whr_ref[...], preferred_element_type=F32) + bhr_ref[...]
        gz = gi_z + jnp.dot(h, whz_ref[...], preferred_element_type=F32) + bhz_ref[...]
        r = _sigmoid(gr)
        z = _sigmoid(gz)
        hn_pre = jnp.dot(h, whn_ref[...], preferred_element_type=F32) + bhn_ref[...]
        n = jnp.tanh(gi_n + r * hn_pre)
        return (1.0 - z) * n + z * h

    h0 = jnp.zeros((BT, 2 * D), F32)
    h = cell(h0, gi_r0, gi_z0, gi_n0)
    c = c0_ref[...] + jnp.dot(h, wfc_ref[...], preferred_element_type=F32) + bfc_ref[...]
    pred_ref[0] = c

    def step(t, carry):
        h, c = carry
        h = cell(h, bir_ref[...], biz_ref[...], bin_ref[...])
        c = c + jnp.dot(h, wfc_ref[...], preferred_element_type=F32) + bfc_ref[...]
        pred_ref[t] = c
        return (h, c)

    lax.fori_loop(1, FUT, step, (h, c))
    p = pred_ref[...]
    flat, valid = _pred_to_idx(p)
    idx_ref[...] = flat
    val_ref[...] = valid


def _refine_kernel(pred_in_ref, rows_ref, valm_ref, h0_ref,
                   wir_ref, wiz_ref, win_ref, whr_ref, whz_ref, whn_ref,
                   bir_ref, biz_ref, bin_ref, bhr_ref, bhz_ref, bhn_ref,
                   wr3_ref, br3_ref,
                   pred_ref, idx_ref, val_ref):
    def step(t, h):
        x = rows_ref[t][:, 0:32] * valm_ref[t]
        gr = jnp.dot(x, wir_ref[...], preferred_element_type=F32) + bir_ref[...] \
            + jnp.dot(h, whr_ref[...], preferred_element_type=F32) + bhr_ref[...]
        gz = jnp.dot(x, wiz_ref[...], preferred_element_type=F32) + biz_ref[...] \
            + jnp.dot(h, whz_ref[...], preferred_element_type=F32) + bhz_ref[...]
        r = _sigmoid(gr)
        z = _sigmoid(gz)
        hn_pre = jnp.dot(h, whn_ref[...], preferred_element_type=F32) + bhn_ref[...]
        n = jnp.tanh(jnp.dot(x, win_ref[...], preferred_element_type=F32)
                     + bin_ref[...] + r * hn_pre)
        return (1.0 - z) * n + z * h

    h = lax.fori_loop(0, FUT, step, h0_ref[...])

    def wstep(t, _):
        r_t = jnp.dot(h, wr3_ref[t], preferred_element_type=F32) + br3_ref[t]
        pred_ref[t] = pred_in_ref[t] + r_t
        return 0

    lax.fori_loop(0, FUT, wstep, 0)
    p = pred_ref[...]
    flat, valid = _pred_to_idx(p)
    idx_ref[...] = flat
    val_ref[...] = valid


def _split3(w, n):
    return w[0:n].T, w[n:2 * n].T, w[2 * n:3 * n].T


def _b3(b, n):
    return b[0:n].reshape(1, n), b[n:2 * n].reshape(1, n), b[2 * n:3 * n].reshape(1, n)


def kernel(past, scene, memory_past, memory_fut, params):
    p = params

    w6 = jnp.transpose(p['W_conv_past'], (2, 1, 0)).reshape(6, 16)
    b16 = p['b_conv_past'].reshape(1, 16)
    e_wir, e_wiz, e_win = _split3(p['W_ih_enc'], D)
    e_whr, e_whz, e_whn = _split3(p['W_hh_enc'], D)
    e_bir, e_biz, e_bin = _b3(p['b_ih_enc'], D)
    e_bhr, e_bhz, e_bhn = _b3(p['b_hh_enc'], D)
    d_wir, d_wiz, d_win = _split3(p['W_ih_dec'], 2 * D)
    d_whr, d_whz, d_whn = _split3(p['W_hh_dec'], 2 * D)
    d_bir, d_biz, d_bin = _b3(p['b_ih_dec'], 2 * D)
    d_bhr, d_bhz, d_bhn = _b3(p['b_hh_dec'], 2 * D)
    s_wir, s_wiz, s_win = _split3(p['W_ih_scn'], D)
    s_whr, s_whz, s_whn = _split3(p['W_hh_scn'], D)
    s_bir, s_biz, s_bin = _b3(p['b_ih_scn'], D)
    s_bhr, s_bhz, s_bhn = _b3(p['b_hh_scn'], D)
    wfc = p['W_fc_out'].T
    bfc = p['b_fc_out'].reshape(1, 2)
    w1 = jnp.transpose(p['W_cs1'], (2, 3, 1, 0)).reshape(100, 16)
    b1 = p['b_cs1'].reshape(1, 16)
    w2 = jnp.transpose(p['W_cs2'], (2, 3, 1, 0)).reshape(5, 80, 32)
    b2 = p['b_cs2'].reshape(1, 32)
    wr3 = jnp.transpose(p['W_refine'].reshape(FUT, 2, D), (0, 2, 1))
    br3 = p['b_refine'].reshape(FUT, 1, 2)

    xt = jnp.transpose(past, (1, 0, 2))
    xp = jnp.concatenate([jnp.zeros((1, B, 2), F32), xt, jnp.zeros((1, B, 2), F32)], 0)
    pat = jnp.concatenate([xp[0:20], xp[1:21], xp[2:22]], axis=2)

    state_past, q_n = pl.pallas_call(
        _enc_kernel,
        out_shape=[jax.ShapeDtypeStruct((B, D), F32),
                   jax.ShapeDtypeStruct((B, D), F32)],
        scratch_shapes=[pltpu.VMEM((20, B, 16), F32)],
    )(pat, w6, b16, e_wir, e_wiz, e_win, e_whr, e_whz, e_whn,
      e_bir, e_biz, e_bin, e_bhr, e_bhz, e_bhn)

    sp = jnp.pad(scene, ((0, 0), (2, 2), (2, 2), (0, 0)))
    phases = jnp.transpose(sp.reshape(B, 92, 2, 92, 2, 4), (0, 2, 4, 1, 3, 5))
    s2 = pl.pallas_call(
        _scene_kernel,
        grid=(B,),
        in_specs=[
            pl.BlockSpec((1, 2, 2, 92, 92, 4), lambda b: (b, 0, 0, 0, 0, 0)),
            pl.BlockSpec((100, 16), lambda b: (0, 0)),
            pl.BlockSpec((1, 16), lambda b: (0, 0)),
            pl.BlockSpec((5, 80, 32), lambda b: (0, 0, 0)),
            pl.BlockSpec((1, 32), lambda b: (0, 0)),
        ],
        out_specs=pl.BlockSpec((PIXP, 128), lambda b: (b, 0)),
        out_shape=jax.ShapeDtypeStruct((B * PIXP, 128), F32),
        scratch_shapes=[pltpu.VMEM((94, 94, 16), F32)],
    )(phases, w1, b1, w2, b2)
    table = s2

    bvals, bidx = pl.pallas_call(
        _sim_kernel,
        grid=(NBLK,),
        in_specs=[
            pl.BlockSpec((B, D), lambda i: (0, 0)),
            pl.BlockSpec((MBLK, D), lambda i: (i, 0)),
        ],
        out_specs=[
            pl.BlockSpec((1, B, NP), lambda i: (i, 0, 0)),
            pl.BlockSpec((1, B, NP), lambda i: (i, 0, 0)),
        ],
        out_shape=[jax.ShapeDtypeStruct((NBLK, B, NP), F32),
                   jax.ShapeDtypeStruct((NBLK, B, NP), jnp.int32)],
    )(q_n, memory_past)
    cv = jnp.transpose(bvals, (1, 0, 2)).reshape(B, NBLK * NP)
    ci = jnp.transpose(bidx, (1, 0, 2)).reshape(B, NBLK * NP)
    topk = pl.pallas_call(
        _merge_kernel,
        out_shape=jax.ShapeDtypeStruct((B, NP), jnp.int32),
    )(cv, ci)

    fidx = jnp.concatenate([topk.reshape(BT), jnp.zeros((512 - BT,), jnp.int32)])
    fut_pad = jnp.pad(memory_fut, ((0, 0), (0, 128 - D)))
    fut_rows = _sc_gather(fut_pad, fidx, 512, 16, 128)
    info_future = fut_rows.reshape(512, 128)[:BT, :D]

    state_rep = jnp.repeat(state_past, NP, axis=0)
    inp0 = jnp.concatenate([state_rep, info_future], axis=1)
    c0 = jnp.repeat(past[:, -1, :], NP, axis=0)
    pred, idxf, valf = pl.pallas_call(
        _dec_kernel,
        out_shape=[jax.ShapeDtypeStruct((FUT, BT, 2), F32),
                   jax.ShapeDtypeStruct((FUT, BT, 1), jnp.int32),
                   jax.ShapeDtypeStruct((FUT, BT, 1), F32)],
    )(inp0, c0, d_wir, d_wiz, d_win, d_whr, d_whz, d_whn,
      d_bir, d_biz, d_bin, d_bhr, d_bhz, d_bhn, wfc, bfc)

    for _ in range(4):
        rows = _sc_gather(table, idxf.reshape(FUT * BT), FUT * BT, 80, 128)
        rows = rows.reshape(FUT, BT, 128)
        pred, idxf, valf = pl.pallas_call(
            _refine_kernel,
            out_shape=[jax.ShapeDtypeStruct((FUT, BT, 2), F32),
                       jax.ShapeDtypeStruct((FUT, BT, 1), jnp.int32),
                       jax.ShapeDtypeStruct((FUT, BT, 1), F32)],
        )(pred, rows, valf, state_rep,
          s_wir, s_wiz, s_win, s_whr, s_whz, s_whn,
          s_bir, s_biz, s_bin, s_bhr, s_bhz, s_bhn, wr3, br3)

    return jnp.transpose(pred, (1, 0, 2)).reshape(B, NP, FUT, 2)


# speedup vs baseline: 1.0587x; 1.0314x over previous
import functools

import jax
import jax.numpy as jnp
from jax import lax
from jax.experimental import pallas as pl
from jax.experimental.pallas import tpu as pltpu
from jax.experimental.pallas import tpu_sc as plsc

F32 = jnp.float32
D = 48
B = 64
NP = 5
FUT = 40
BT = B * NP
MEM = 50000
MBLK = 2000
NBLK = MEM // MBLK
HW = 90
PIX = HW * HW
PIXP = PIX + 4

SC_NC, SC_NS = 2, 16
NW = SC_NC * SC_NS


def _sigmoid(x):
    return jax.nn.sigmoid(x)


def _sc_gather(table, idx1d, total, chunk, dfeat):
    per_w = total // NW
    nchunk = per_w // chunk
    mesh = plsc.VectorSubcoreMesh(core_axis_name="c", subcore_axis_name="s")

    @functools.partial(
        pl.kernel,
        mesh=mesh,
        out_type=jax.ShapeDtypeStruct((total, dfeat), F32),
        scratch_types=[
            pltpu.VMEM((per_w,), jnp.int32),
            pltpu.VMEM((per_w, dfeat), F32),
            pltpu.SemaphoreType.DMA,
        ],
    )
    def k(table_hbm, idx_hbm, out_hbm, idx_v, rows_v, sem):
        wid = lax.axis_index("s") * SC_NC + lax.axis_index("c")
        base = wid * per_w
        pltpu.sync_copy(idx_hbm.at[pl.ds(base, per_w)], idx_v)
        copies = []
        for j in range(nchunk):
            copies.append(
                pltpu.async_copy(
                    table_hbm.at[idx_v.at[pl.ds(j * chunk, chunk)]],
                    rows_v.at[pl.ds(j * chunk, chunk)], sem)
            )
        for c in copies:
            c.wait()
        pltpu.sync_copy(rows_v, out_hbm.at[pl.ds(base, per_w)])

    return k(table, idx1d)


def _enc_kernel(pat_ref, w6_ref, b16_ref,
                wir_ref, wiz_ref, win_ref, whr_ref, whz_ref, whn_ref,
                bir_ref, biz_ref, bin_ref, bhr_ref, bhz_ref, bhn_ref,
                h_out, q_out, story_ref):
    pat = pat_ref[...].reshape(20 * B, 6)
    story = jax.nn.relu(
        jnp.dot(pat, w6_ref[...], preferred_element_type=F32) + b16_ref[...]
    )
    story_ref[...] = story.reshape(20, B, 16)

    def step(t, h):
        x = story_ref[t]
        gr = jnp.dot(x, wir_ref[...], preferred_element_type=F32) + bir_ref[...] \
            + jnp.dot(h, whr_ref[...], preferred_element_type=F32) + bhr_ref[...]
        gz = jnp.dot(x, wiz_ref[...], preferred_element_type=F32) + biz_ref[...] \
            + jnp.dot(h, whz_ref[...], preferred_element_type=F32) + bhz_ref[...]
        r = _sigmoid(gr)
        z = _sigmoid(gz)
        hn_pre = jnp.dot(h, whn_ref[...], preferred_element_type=F32) + bhn_ref[...]
        n = jnp.tanh(jnp.dot(x, win_ref[...], preferred_element_type=F32)
                     + bin_ref[...] + r * hn_pre)
        return (1.0 - z) * n + z * h

    h = lax.fori_loop(0, 20, step, jnp.zeros((B, D), F32))
    h_out[...] = h
    nrm = jnp.sqrt(jnp.sum(h * h, axis=1, keepdims=True))
    q_out[...] = h / jnp.maximum(nrm, 1e-12)


def _scene_kernel(ph_ref, w1_ref, b1_ref, w2_ref, b2_ref, out_ref, s1p_ref):
    pieces = []
    for dy in range(5):
        ay, by = dy // 2, dy % 2
        for dx in range(5):
            ax, bx = dx // 2, dx % 2
            sl = ph_ref[0, by * 2 + bx, ay:ay + HW, ax:ax + HW, :]
            pieces.append(sl.reshape(PIX, 4))
    patches = jnp.concatenate(pieces, axis=1)
    s1 = jax.nn.relu(
        jnp.dot(patches, w1_ref[...], preferred_element_type=F32) + b1_ref[...]
    )
    s1p_ref[...] = jnp.zeros((94, 94, 16), F32)
    s1p_ref[2:92, 2:92, :] = s1.reshape(HW, HW, 16)
    acc = jnp.zeros((PIX, 32), F32) + b2_ref[...]
    for dy in range(5):
        row_pieces = []
        for dx in range(5):
            row_pieces.append(s1p_ref[dy:dy + HW, dx:dx + HW, :].reshape(PIX, 16))
        patch2 = jnp.concatenate(row_pieces, axis=1)
        acc = acc + jnp.dot(patch2, w2_ref[dy], preferred_element_type=F32)
    out_ref[...] = jnp.concatenate(
        [jnp.concatenate([jax.nn.relu(acc), jnp.zeros((4, 32), F32)], axis=0),
         jnp.zeros((PIXP, 96), F32)], axis=1)


def _sim_kernel(q_ref, mem_ref, vals_ref, idx_ref):
    m = mem_ref[...]
    ss = jnp.sum(m * m, axis=1, keepdims=True)
    mn = m / jnp.maximum(jnp.sqrt(ss), 1e-12)
    s = lax.dot_general(q_ref[...], mn, (((1,), (1,)), ((), ())),
                        preferred_element_type=F32)
    iota = lax.broadcasted_iota(jnp.int32, (B, MBLK), 1)
    base = pl.program_id(0) * MBLK
    vs, ids = [], []
    for _ in range(NP):
        mx = jnp.max(s, axis=1, keepdims=True)
        is_m = s == mx
        am = jnp.min(jnp.where(is_m, iota, jnp.int32(1 << 30)), axis=1,
                     keepdims=True)
        vs.append(mx)
        ids.append(am + base)
        s = jnp.where(iota == am, -jnp.inf, s)
    vals_ref[0] = jnp.concatenate(vs, axis=1)
    idx_ref[0] = jnp.concatenate(ids, axis=1)


def _merge_kernel(cv_ref, ci_ref, idx_ref):
    s = cv_ref[...]
    ci = ci_ref[...]
    ncand = NBLK * NP
    iota = lax.broadcasted_iota(jnp.int32, (B, ncand), 1)
    ids = []
    for _ in range(NP):
        mx = jnp.max(s, axis=1, keepdims=True)
        is_m = s == mx
        p = jnp.min(jnp.where(is_m, iota, jnp.int32(1 << 30)), axis=1,
                    keepdims=True)
        hit = iota == p
        ids.append(jnp.sum(jnp.where(hit, ci, 0), axis=1, keepdims=True))
        s = jnp.where(hit, -jnp.inf, s)
    idx_ref[...] = jnp.concatenate(ids, axis=1)


def _pred_to_idx(p):
    ix = (p[:, :, 0:1] + 89.0) * 0.5
    iy = (p[:, :, 1:2] + 89.0) * 0.5
    ixn = jnp.round(ix).astype(jnp.int32)
    iyn = jnp.round(iy).astype(jnp.int32)
    valid = ((ixn >= 0) & (ixn < HW) & (iyn >= 0) & (iyn < HW))
    ixc = jnp.clip(ixn, 0, HW - 1)
    iyc = jnp.clip(iyn, 0, HW - 1)
    boff = (lax.broadcasted_iota(jnp.int32, (FUT, BT, 1), 1) // NP) * PIXP
    flat = boff + iyc * HW + ixc
    return flat, valid.astype(F32)


def _dec_kernel(inp_ref, c0_ref,
                wir_ref, wiz_ref, win_ref, whr_ref, whz_ref, whn_ref,
                bir_ref, biz_ref, bin_ref, bhr_ref, bhz_ref, bhn_ref,
                wfc_ref, bfc_ref,
                pred_ref, idx_ref, val_ref):
    inp = inp_ref[...]
    gi_r0 = jnp.dot(inp, wir_ref[...], preferred_element_type=F32) + bir_ref[...]
    gi_z0 = jnp.dot(inp, wiz_ref[...], preferred_element_type=F32) + biz_ref[...]
    gi_n0 = jnp.dot(inp, win_ref[...], preferred_element_type=F32) + bin_ref[...]

    def cell(h, gi_r, gi_z, gi_n):
        gr = gi_r + jnp.dot(h, whr_ref[...], preferred_element_type=F32) + bhr_ref[...]
        gz = gi_z + jnp.dot(h, whz_ref[...], preferred_element_type=F32) + bhz_ref[...]
        r = _sigmoid(gr)
        z = _sigmoid(gz)
        hn_pre = jnp.dot(h, whn_ref[...], preferred_element_type=F32) + bhn_ref[...]
        n = jnp.tanh(gi_n + r * hn_pre)
        return (1.0 - z) * n + z * h

    h0 = jnp.zeros((BT, 2 * D), F32)
    h = cell(h0, gi_r0, gi_z0, gi_n0)
    c = c0_ref[...] + jnp.dot(h, wfc_ref[...], preferred_element_type=F32) + bfc_ref[...]
    pred_ref[0] = c

    def step(t, carry):
        h, c = carry
        h = cell(h, bir_ref[...], biz_ref[...], bin_ref[...])
        c = c + jnp.dot(h, wfc_ref[...], preferred_element_type=F32) + bfc_ref[...]
        pred_ref[t] = c
        return (h, c)

    lax.fori_loop(1, FUT, step, (h, c))
    p = pred_ref[...]
    flat, valid = _pred_to_idx(p)
    idx_ref[...] = flat
    val_ref[...] = valid


def _refine_kernel(pred_in_ref, rows_ref, valm_ref, h0_ref,
                   wir_ref, wiz_ref, win_ref, whr_ref, whz_ref, whn_ref,
                   bir_ref, biz_ref, bin_ref, bhr_ref, bhz_ref, bhn_ref,
                   wr3_ref, br3_ref,
                   pred_ref, idx_ref, val_ref):
    def step(t, h):
        x = rows_ref[t][:, 0:32] * valm_ref[t]
        gr = jnp.dot(x, wir_ref[...], preferred_element_type=F32) + bir_ref[...] \
            + jnp.dot(h, whr_ref[...], preferred_element_type=F32) + bhr_ref[...]
        gz = jnp.dot(x, wiz_ref[...], preferred_element_type=F32) + biz_ref[...] \
            + jnp.dot(h, whz_ref[...], preferred_element_type=F32) + bhz_ref[...]
        r = _sigmoid(gr)
        z = _sigmoid(gz)
        hn_pre = jnp.dot(h, whn_ref[...], preferred_element_type=F32) + bhn_ref[...]
        n = jnp.tanh(jnp.dot(x, win_ref[...], preferred_element_type=F32)
                     + bin_ref[...] + r * hn_pre)
        return (1.0 - z) * n + z * h

    h = lax.fori_loop(0, FUT, step, h0_ref[...])

    def wstep(t, _):
        r_t = jnp.dot(h, wr3_ref[t], preferred_element_type=F32) + br3_ref[t]
        pred_ref[t] = pred_in_ref[t] + r_t
        return 0

    lax.fori_loop(0, FUT, wstep, 0)
    p = pred_ref[...]
    flat, valid = _pred_to_idx(p)
    idx_ref[...] = flat
    val_ref[...] = valid


def _split3(w, n):
    return w[0:n].T, w[n:2 * n].T, w[2 * n:3 * n].T


def _b3(b, n):
    return b[0:n].reshape(1, n), b[n:2 * n].reshape(1, n), b[2 * n:3 * n].reshape(1, n)


def kernel(past, scene, memory_past, memory_fut, params):
    p = params

    w6 = jnp.transpose(p['W_conv_past'], (2, 1, 0)).reshape(6, 16)
    b16 = p['b_conv_past'].reshape(1, 16)
    e_wir, e_wiz, e_win = _split3(p['W_ih_enc'], D)
    e_whr, e_whz, e_whn = _split3(p['W_hh_enc'], D)
    e_bir, e_biz, e_bin = _b3(p['b_ih_enc'], D)
    e_bhr, e_bhz, e_bhn = _b3(p['b_hh_enc'], D)
    d_wir, d_wiz, d_win = _split3(p['W_ih_dec'], 2 * D)
    d_whr, d_whz, d_whn = _split3(p['W_hh_dec'], 2 * D)
    d_bir, d_biz, d_bin = _b3(p['b_ih_dec'], 2 * D)
    d_bhr, d_bhz, d_bhn = _b3(p['b_hh_dec'], 2 * D)
    s_wir, s_wiz, s_win = _split3(p['W_ih_scn'], D)
    s_whr, s_whz, s_whn = _split3(p['W_hh_scn'], D)
    s_bir, s_biz, s_bin = _b3(p['b_ih_scn'], D)
    s_bhr, s_bhz, s_bhn = _b3(p['b_hh_scn'], D)
    wfc = p['W_fc_out'].T
    bfc = p['b_fc_out'].reshape(1, 2)
    w1 = jnp.transpose(p['W_cs1'], (2, 3, 1, 0)).reshape(100, 16)
    b1 = p['b_cs1'].reshape(1, 16)
    w2 = jnp.transpose(p['W_cs2'], (2, 3, 1, 0)).reshape(5, 80, 32)
    b2 = p['b_cs2'].reshape(1, 32)
    wr3 = jnp.transpose(p['W_refine'].reshape(FUT, 2, D), (0, 2, 1))
    br3 = p['b_refine'].reshape(FUT, 1, 2)

    xt = jnp.transpose(past, (1, 0, 2))
    xp = jnp.concatenate([jnp.zeros((1, B, 2), F32), xt, jnp.zeros((1, B, 2), F32)], 0)
    pat = jnp.concatenate([xp[0:20], xp[1:21], xp[2:22]], axis=2)

    state_past, q_n = pl.pallas_call(
        _enc_kernel,
        out_shape=[jax.ShapeDtypeStruct((B, D), F32),
                   jax.ShapeDtypeStruct((B, D), F32)],
        scratch_shapes=[pltpu.VMEM((20, B, 16), F32)],
    )(pat, w6, b16, e_wir, e_wiz, e_win, e_whr, e_whz, e_whn,
      e_bir, e_biz, e_bin, e_bhr, e_bhz, e_bhn)

    sp = jnp.pad(scene, ((0, 0), (2, 2), (2, 2), (0, 0)))
    phases = jnp.stack(
        [sp[:, by::2, bx::2, :] for by in (0, 1) for bx in (0, 1)], axis=1)
    s2 = pl.pallas_call(
        _scene_kernel,
        grid=(B,),
        in_specs=[
            pl.BlockSpec((1, 4, 92, 92, 4), lambda b: (b, 0, 0, 0, 0)),
            pl.BlockSpec((100, 16), lambda b: (0, 0)),
            pl.BlockSpec((1, 16), lambda b: (0, 0)),
            pl.BlockSpec((5, 80, 32), lambda b: (0, 0, 0)),
            pl.BlockSpec((1, 32), lambda b: (0, 0)),
        ],
        out_specs=pl.BlockSpec((PIXP, 128), lambda b: (b, 0)),
        out_shape=jax.ShapeDtypeStruct((B * PIXP, 128), F32),
        scratch_shapes=[pltpu.VMEM((94, 94, 16), F32)],
    )(phases, w1, b1, w2, b2)
    table = s2

    bvals, bidx = pl.pallas_call(
        _sim_kernel,
        grid=(NBLK,),
        in_specs=[
            pl.BlockSpec((B, D), lambda i: (0, 0)),
            pl.BlockSpec((MBLK, D), lambda i: (i, 0)),
        ],
        out_specs=[
            pl.BlockSpec((1, B, NP), lambda i: (i, 0, 0)),
            pl.BlockSpec((1, B, NP), lambda i: (i, 0, 0)),
        ],
        out_shape=[jax.ShapeDtypeStruct((NBLK, B, NP), F32),
                   jax.ShapeDtypeStruct((NBLK, B, NP), jnp.int32)],
    )(q_n, memory_past)
    cv = jnp.transpose(bvals, (1, 0, 2)).reshape(B, NBLK * NP)
    ci = jnp.transpose(bidx, (1, 0, 2)).reshape(B, NBLK * NP)
    topk = pl.pallas_call(
        _merge_kernel,
        out_shape=jax.ShapeDtypeStruct((B, NP), jnp.int32),
    )(cv, ci)

    fidx = jnp.concatenate([topk.reshape(BT), jnp.zeros((512 - BT,), jnp.int32)])
    fut_pad = jnp.pad(memory_fut, ((0, 0), (0, 128 - D)))
    fut_rows = _sc_gather(fut_pad, fidx, 512, 16, 128)
    info_future = fut_rows.reshape(512, 128)[:BT, :D]

    state_rep = jnp.repeat(state_past, NP, axis=0)
    inp0 = jnp.concatenate([state_rep, info_future], axis=1)
    c0 = jnp.repeat(past[:, -1, :], NP, axis=0)
    pred, idxf, valf = pl.pallas_call(
        _dec_kernel,
        out_shape=[jax.ShapeDtypeStruct((FUT, BT, 2), F32),
                   jax.ShapeDtypeStruct((FUT, BT, 1), jnp.int32),
                   jax.ShapeDtypeStruct((FUT, BT, 1), F32)],
    )(inp0, c0, d_wir, d_wiz, d_win, d_whr, d_whz, d_whn,
      d_bir, d_biz, d_bin, d_bhr, d_bhz, d_bhn, wfc, bfc)

    for _ in range(4):
        rows = _sc_gather(table, idxf.reshape(FUT * BT), FUT * BT, 80, 128)
        rows = rows.reshape(FUT, BT, 128)
        pred, idxf, valf = pl.pallas_call(
            _refine_kernel,
            out_shape=[jax.ShapeDtypeStruct((FUT, BT, 2), F32),
                       jax.ShapeDtypeStruct((FUT, BT, 1), jnp.int32),
                       jax.ShapeDtypeStruct((FUT, BT, 1), F32)],
        )(pred, rows, valf, state_rep,
          s_wir, s_wiz, s_win, s_whr, s_whz, s_whn,
          s_bir, s_biz, s_bin, s_bhr, s_bhz, s_bhn, wr3, br3)

    return jnp.transpose(pred, (1, 0, 2)).reshape(B, NP, FUT, 2)
